# baseline (speedup 1.0000x reference)
import functools

import jax
import jax.numpy as jnp
from jax import lax
from jax.experimental import pallas as pl
from jax.experimental.pallas import tpu as pltpu
from jax.experimental.pallas import tpu_sc as plsc

D = 64
BATCH = 4096
SEQ = 200
HALF0 = 104
HALF1 = 96

_info = plsc.get_sparse_core_info()
NC, NS, L = _info.num_cores, _info.num_subcores, _info.num_lanes
NW = NC * NS
SEQS_PER_W = BATCH // NW


def _lane_broadcast(v, j):
    idx = jnp.full((L, 1), j, jnp.int32)
    return lax.gather(
        v, idx,
        lax.GatherDimensionNumbers(
            offset_dims=(), collapsed_slice_dims=(0,), start_index_map=(0,)),
        slice_sizes=(1,),
        mode=lax.GatherScatterMode.PROMISE_IN_BOUNDS)


def _body(x_hbm, table_hbm, pos_hbm, out_hbm, idx_v, rows_v, pos_v, sem):
    wid = lax.axis_index("s") * NC + lax.axis_index("c")
    pltpu.sync_copy(pos_hbm, pos_v)

    def seq_body(g, carry):
        seq = wid * SEQS_PER_W + g
        base = seq * SEQ
        pltpu.sync_copy(x_hbm.at[pl.ds(base, SEQ)], idx_v)
        cp0 = pltpu.async_copy(
            table_hbm.at[idx_v.at[pl.ds(0, HALF0)]],
            rows_v.at[pl.ds(0, HALF0), :], sem)
        cp1 = pltpu.async_copy(
            table_hbm.at[idx_v.at[pl.ds(HALF0, HALF1)]],
            rows_v.at[pl.ds(HALF0, HALF1), :], sem)
        cp0.wait()
        cp1.wait()

        def do_rows(ivec, r0, lanes):
            mvec = jnp.where(ivec == 2, 0.0, 1.0).astype(jnp.float32)
            for j in lanes:
                m = _lane_broadcast(mvec, j)
                r = r0 + j
                for q in range(D // L):
                    sl = pl.ds(q * L, L)
                    rows_v[r, sl] = rows_v[r, sl] * m + pos_v[r, sl]

        def grp_body(g2, c):
            do_rows(idx_v[pl.ds(g2 * L, L)], g2 * L, range(L))
            return c

        lax.fori_loop(0, SEQ // L, grp_body, 0)
        do_rows(idx_v[pl.ds(SEQ - L, L)], SEQ - L, range(L - (SEQ % L), L))
        pltpu.sync_copy(rows_v, out_hbm.at[pl.ds(base, SEQ), :])
        return carry

    lax.fori_loop(0, SEQS_PER_W, seq_body, 0)


@jax.jit
def _run(xf, table, pos_enc):
    fn = pl.kernel(
        _body,
        mesh=plsc.VectorSubcoreMesh(core_axis_name="c", subcore_axis_name="s"),
        compiler_params=pltpu.CompilerParams(use_tc_tiling_on_sc=False),
        out_type=jax.ShapeDtypeStruct((BATCH * SEQ, D), jnp.float32),
        scratch_types=[
            pltpu.VMEM((SEQ,), jnp.int32),
            pltpu.VMEM((SEQ, D), jnp.float32),
            pltpu.VMEM((SEQ, D), jnp.float32),
            pltpu.SemaphoreType.DMA,
        ],
    )
    return fn(xf, table, pos_enc)


def kernel(x, table, pos_enc):
    out = _run(x.reshape(BATCH * SEQ), table, pos_enc)
    return out.reshape(BATCH, SEQ, D)


# speedup vs baseline: 1.1597x; 1.1597x over previous
import jax
import jax.numpy as jnp
from jax import lax
from jax.experimental import pallas as pl
from jax.experimental.pallas import tpu as pltpu
from jax.experimental.pallas import tpu_sc as plsc

D = 64
BATCH = 4096
SEQ = 200
HALF0 = 104
HALF1 = 96

_info = plsc.get_sparse_core_info()
NC, NS, L = _info.num_cores, _info.num_subcores, _info.num_lanes
NW = NC * NS
SEQS_PER_W = BATCH // NW
CHUNK_BYTES = SEQ * D * 4


def _lane_broadcast(v, j):
    idx = jnp.full((L, 1), j, jnp.int32)
    return lax.gather(
        v, idx,
        lax.GatherDimensionNumbers(
            offset_dims=(), collapsed_slice_dims=(0,), start_index_map=(0,)),
        slice_sizes=(1,),
        mode=lax.GatherScatterMode.PROMISE_IN_BOUNDS)


def _body(x_hbm, table_hbm, pos_hbm, out_hbm,
          idx_all, rows0, rows1, pos_v, gsem0, gsem1, ssem0, ssem1):
    wid = lax.axis_index("s") * NC + lax.axis_index("c")
    wbase = wid * SEQS_PER_W * SEQ
    pltpu.sync_copy(pos_hbm, pos_v)
    pltpu.sync_copy(x_hbm.at[pl.ds(wbase, SEQS_PER_W * SEQ)], idx_all)

    rows = (rows0, rows1)
    gsem = (gsem0, gsem1)
    ssem = (ssem0, ssem1)

    def issue_gather(c, b):
        off = c * SEQ
        pltpu.async_copy(table_hbm.at[idx_all.at[pl.ds(off, HALF0)]],
                         rows[b].at[pl.ds(0, HALF0), :], gsem[b])
        pltpu.async_copy(table_hbm.at[idx_all.at[pl.ds(off + HALF0, HALF1)]],
                         rows[b].at[pl.ds(HALF0, HALF1), :], gsem[b])

    def wait_gather(b):
        pltpu.make_async_copy(out_hbm.at[pl.ds(0, SEQ), :], rows[b],
                              gsem[b]).wait()

    def wait_scatter(b):
        pltpu.make_async_copy(rows[b], out_hbm.at[pl.ds(0, SEQ), :],
                              ssem[b]).wait()

    def compute(c, b):
        coff = c * SEQ

        def do_rows(ivec, r0, lanes):
            mvec = jnp.where(ivec == 2, 0.0, 1.0).astype(jnp.float32)
            for j in lanes:
                m = _lane_broadcast(mvec, j)
                r = r0 + j
                for q in range(D // L):
                    sl = pl.ds(q * L, L)
                    rows[b][r, sl] = rows[b][r, sl] * m + pos_v[r, sl]

        def grp_body(g2, cc):
            do_rows(idx_all[pl.ds(coff + g2 * L, L)], g2 * L, range(L))
            return cc

        lax.fori_loop(0, SEQ // L, grp_body, 0)
        do_rows(idx_all[pl.ds(coff + SEQ - L, L)], SEQ - L,
                range(L - SEQ % L, L))

    issue_gather(0, 0)

    def pair_body(gi, carry):
        for b in (0, 1):
            c = gi * 2 + b
            nb = 1 - b

            @pl.when(c >= 1)
            def _():
                wait_scatter(nb)

            @pl.when(c + 1 < SEQS_PER_W)
            def _():
                issue_gather(c + 1, nb)

            wait_gather(b)
            compute(c, b)
            pltpu.async_copy(rows[b],
                             out_hbm.at[pl.ds(wbase + c * SEQ, SEQ), :],
                             ssem[b])
        return carry

    lax.fori_loop(0, SEQS_PER_W // 2, pair_body, 0)
    wait_scatter(1)


@jax.jit
def _run(xf, table, pos_enc):
    fn = pl.kernel(
        _body,
        mesh=plsc.VectorSubcoreMesh(core_axis_name="c", subcore_axis_name="s"),
        compiler_params=pltpu.CompilerParams(use_tc_tiling_on_sc=False),
        out_type=jax.ShapeDtypeStruct((BATCH * SEQ, D), jnp.float32),
        scratch_types=[
            pltpu.VMEM((SEQS_PER_W * SEQ,), jnp.int32),
            pltpu.VMEM((SEQ, D), jnp.float32),
            pltpu.VMEM((SEQ, D), jnp.float32),
            pltpu.VMEM((SEQ, D), jnp.float32),
            pltpu.SemaphoreType.DMA,
            pltpu.SemaphoreType.DMA,
            pltpu.SemaphoreType.DMA,
            pltpu.SemaphoreType.DMA,
        ],
    )
    return fn(xf, table, pos_enc)


def kernel(x, table, pos_enc):
    out = _run(x.reshape(BATCH * SEQ), table, pos_enc)
    return out.reshape(BATCH, SEQ, D)
